# baseline (speedup 1.0000x reference)
import functools

import numpy as np
import jax
import jax.numpy as jnp
from jax.experimental import pallas as pl
from jax.experimental.pallas import tpu as pltpu

C_IN, C_OUT, KS, POOL = 3, 8, 3, 4
H = W = 48
HP = WP = H // POOL
N_CLASSES = 43
LANES_IN = C_IN * W
LANES_CONV = C_OUT * W
POOLED = C_OUT * WP
CAND_STRIDE = 128

_shift_np = np.zeros((KS, W, W), np.float32)
for _kx in range(KS):
    for _wo in range(W):
        _wi = _wo + _kx - 1
        if 0 <= _wi < W:
            _shift_np[_kx, _wi, _wo] = 1.0

_selc_np = np.zeros((LANES_CONV, 4 * CAND_STRIDE), np.float32)
for _j in range(POOL):
    for _co in range(C_OUT):
        for _wp in range(WP):
            _selc_np[_co * W + POOL * _wp + _j,
                     CAND_STRIDE * _j + _co * WP + _wp] = 1.0


def _cnn_kernel(x_ref, wky_ref, selc_ref, b96_ref, wf_ref, bf_ref, out_ref,
                xt_ref):
    bblk = out_ref.shape[0]
    nb = HP * bblk

    for j in range(POOL):
        for hp in range(HP):
            row = j * nb + hp * bblk
            for ci in range(C_IN):
                xt_ref[row:row + bblk, ci * W:(ci + 1) * W] = (
                    x_ref[:, ci, POOL * hp + j, :])

    wky = wky_ref[...]

    def zmm(ky, j):
        return jnp.dot(xt_ref[j * nb:(j + 1) * nb, :], wky[ky],
                       preferred_element_type=jnp.float32)

    zpad = jnp.zeros((bblk, LANES_CONV), jnp.float32)
    z00 = zmm(0, 0)
    z23 = zmm(2, 3)
    conv0 = jnp.concatenate([zpad, zmm(0, 3)[:-bblk]], axis=0) + zmm(1, 0) \
        + zmm(2, 1)
    conv1 = z00 + zmm(1, 1) + zmm(2, 2)
    conv2 = zmm(0, 1) + zmm(1, 2) + z23
    conv3 = zmm(0, 2) + zmm(1, 3) \
        + jnp.concatenate([zmm(2, 0)[bblk:], zpad], axis=0)

    r12 = jnp.maximum(jnp.maximum(conv0, conv1),
                      jnp.maximum(conv2, conv3))

    cand = jnp.dot(r12, selc_ref[...], preferred_element_type=jnp.float32)
    pooled = jnp.maximum(
        jnp.maximum(cand[:, 0:POOLED],
                    cand[:, CAND_STRIDE:CAND_STRIDE + POOLED]),
        jnp.maximum(cand[:, 2 * CAND_STRIDE:2 * CAND_STRIDE + POOLED],
                    cand[:, 3 * CAND_STRIDE:3 * CAND_STRIDE + POOLED]))
    act = jnp.maximum(pooled + b96_ref[...], 0.0)

    logits = jnp.broadcast_to(bf_ref[...], (bblk, N_CLASSES))
    wf = wf_ref[...]
    for hp in range(HP):
        logits = logits + jnp.dot(act[hp * bblk:(hp + 1) * bblk], wf[hp],
                                  preferred_element_type=jnp.float32)
    out_ref[...] = logits


@functools.partial(jax.jit, static_argnames=("block_b",))
def _forward(x, conv_w, conv_b, fc_w, fc_b, block_b=64):
    B = x.shape[0]
    bblk = block_b
    while B % bblk:
        bblk //= 2
    n_steps = B // bblk

    wky = jnp.einsum("xab,ocyx->ycaob", jnp.asarray(_shift_np),
                     conv_w.astype(jnp.float32))
    wky = wky.reshape(KS, LANES_IN, LANES_CONV)

    b96 = jnp.repeat(conv_b.astype(jnp.float32), WP).reshape(1, POOLED)
    wf = (fc_w.astype(jnp.float32).reshape(N_CLASSES, C_OUT, HP, WP)
          .transpose(2, 1, 3, 0).reshape(HP, POOLED, N_CLASSES))
    bf2 = fc_b.astype(jnp.float32).reshape(1, N_CLASSES)

    return pl.pallas_call(
        _cnn_kernel,
        out_shape=jax.ShapeDtypeStruct((B, N_CLASSES), jnp.float32),
        grid=(n_steps,),
        in_specs=[
            pl.BlockSpec((bblk, C_IN, H, W), lambda s: (s, 0, 0, 0)),
            pl.BlockSpec((KS, LANES_IN, LANES_CONV), lambda s: (0, 0, 0)),
            pl.BlockSpec((LANES_CONV, 4 * CAND_STRIDE), lambda s: (0, 0)),
            pl.BlockSpec((1, POOLED), lambda s: (0, 0)),
            pl.BlockSpec((HP, POOLED, N_CLASSES), lambda s: (0, 0, 0)),
            pl.BlockSpec((1, N_CLASSES), lambda s: (0, 0)),
        ],
        out_specs=pl.BlockSpec((bblk, N_CLASSES), lambda s: (s, 0)),
        scratch_shapes=[pltpu.VMEM((POOL * HP * bblk, LANES_IN), jnp.float32)],
        compiler_params=pltpu.CompilerParams(
            dimension_semantics=("parallel",)),
    )(x.astype(jnp.float32), wky, jnp.asarray(_selc_np), b96, wf, bf2)


def kernel(x, conv_w, conv_b, fc_w, fc_b):
    return _forward(x, conv_w, conv_b, fc_w, fc_b)


# speedup vs baseline: 1.4977x; 1.4977x over previous
import functools

import numpy as np
import jax
import jax.numpy as jnp
from jax.experimental import pallas as pl
from jax.experimental.pallas import tpu as pltpu

C_IN, C_OUT, KS, POOL = 3, 8, 3, 4
H = W = 48
HP = WP = H // POOL
N_CLASSES = 43
LANES_IN = C_IN * W
LANES_CONV = C_OUT * W
POOLED = C_OUT * WP
CAND_STRIDE = 128

_shift_np = np.zeros((KS, W, W), np.float32)
for _kx in range(KS):
    for _wo in range(W):
        _wi = _wo + _kx - 1
        if 0 <= _wi < W:
            _shift_np[_kx, _wi, _wo] = 1.0

_selc_np = np.zeros((LANES_CONV, 4 * CAND_STRIDE), np.float32)
for _j in range(POOL):
    for _co in range(C_OUT):
        for _wp in range(WP):
            _selc_np[_co * W + POOL * _wp + _j,
                     CAND_STRIDE * _j + _co * WP + _wp] = 1.0


def _cnn_kernel(xr_ref, wky_ref, selc_ref, b96_ref, wf_ref, bf_ref, out_ref):
    bblk = out_ref.shape[0]
    rows = H * bblk
    xt = xr_ref[...].reshape(rows, LANES_IN)
    wky = wky_ref[...]

    zpad = jnp.zeros((bblk, LANES_CONV), jnp.float32)
    conv = jnp.dot(xt, wky[1], preferred_element_type=jnp.float32)
    z0 = jnp.dot(xt, wky[0], preferred_element_type=jnp.float32)
    conv = conv + jnp.concatenate([zpad, z0[:rows - bblk]], axis=0)
    z2 = jnp.dot(xt, wky[2], preferred_element_type=jnp.float32)
    conv = conv + jnp.concatenate([z2[bblk:], zpad], axis=0)

    r12 = jnp.concatenate(
        [jnp.maximum(
            jnp.maximum(conv[(4 * hp) * bblk:(4 * hp + 1) * bblk],
                        conv[(4 * hp + 1) * bblk:(4 * hp + 2) * bblk]),
            jnp.maximum(conv[(4 * hp + 2) * bblk:(4 * hp + 3) * bblk],
                        conv[(4 * hp + 3) * bblk:(4 * hp + 4) * bblk]))
         for hp in range(HP)], axis=0)

    cand = jnp.dot(r12, selc_ref[...], preferred_element_type=jnp.float32)
    pooled = jnp.maximum(
        jnp.maximum(cand[:, 0:POOLED],
                    cand[:, CAND_STRIDE:CAND_STRIDE + POOLED]),
        jnp.maximum(cand[:, 2 * CAND_STRIDE:2 * CAND_STRIDE + POOLED],
                    cand[:, 3 * CAND_STRIDE:3 * CAND_STRIDE + POOLED]))
    act = jnp.maximum(pooled + b96_ref[...], 0.0)

    logits = jnp.broadcast_to(bf_ref[...], (bblk, N_CLASSES))
    wf = wf_ref[...]
    for hp in range(HP):
        logits = logits + jnp.dot(act[hp * bblk:(hp + 1) * bblk], wf[hp],
                                  preferred_element_type=jnp.float32)
    out_ref[...] = logits


@functools.partial(jax.jit, static_argnames=("block_b",))
def _forward(x, conv_w, conv_b, fc_w, fc_b, block_b=128):
    B = x.shape[0]
    bblk = block_b
    while B % bblk:
        bblk //= 2
    n_steps = B // bblk

    xr = (x.astype(jnp.float32).transpose(2, 0, 1, 3)
          .reshape(H, B, LANES_IN).astype(jnp.bfloat16))

    wky = jnp.einsum("xab,ocyx->ycaob", jnp.asarray(_shift_np),
                     conv_w.astype(jnp.float32))
    wky = wky.reshape(KS, LANES_IN, LANES_CONV).astype(jnp.bfloat16)

    b96 = jnp.repeat(conv_b.astype(jnp.float32), WP).reshape(1, POOLED)
    wf = (fc_w.astype(jnp.float32).reshape(N_CLASSES, C_OUT, HP, WP)
          .transpose(2, 1, 3, 0).reshape(HP, POOLED, N_CLASSES))
    bf2 = fc_b.astype(jnp.float32).reshape(1, N_CLASSES)

    return pl.pallas_call(
        _cnn_kernel,
        out_shape=jax.ShapeDtypeStruct((B, N_CLASSES), jnp.float32),
        grid=(n_steps,),
        in_specs=[
            pl.BlockSpec((H, bblk, LANES_IN), lambda s: (0, s, 0)),
            pl.BlockSpec((KS, LANES_IN, LANES_CONV), lambda s: (0, 0, 0)),
            pl.BlockSpec((LANES_CONV, 4 * CAND_STRIDE), lambda s: (0, 0)),
            pl.BlockSpec((1, POOLED), lambda s: (0, 0)),
            pl.BlockSpec((HP, POOLED, N_CLASSES), lambda s: (0, 0, 0)),
            pl.BlockSpec((1, N_CLASSES), lambda s: (0, 0)),
        ],
        out_specs=pl.BlockSpec((bblk, N_CLASSES), lambda s: (s, 0)),
        compiler_params=pltpu.CompilerParams(
            dimension_semantics=("parallel",)),
    )(xr, wky, jnp.asarray(_selc_np), b96, wf, bf2)


def kernel(x, conv_w, conv_b, fc_w, fc_b):
    return _forward(x, conv_w, conv_b, fc_w, fc_b)


# speedup vs baseline: 1.4987x; 1.0006x over previous
import functools

import numpy as np
import jax
import jax.numpy as jnp
from jax.experimental import pallas as pl
from jax.experimental.pallas import tpu as pltpu

C_IN, C_OUT, KS, POOL = 3, 8, 3, 4
H = W = 48
HP = WP = H // POOL
N_CLASSES = 43
LANES_IN = C_IN * W
LANES_CONV = C_OUT * W
POOLED = C_OUT * WP
CAND_STRIDE = 128

_shift_np = np.zeros((KS, W, W), np.float32)
for _kx in range(KS):
    for _wo in range(W):
        _wi = _wo + _kx - 1
        if 0 <= _wi < W:
            _shift_np[_kx, _wi, _wo] = 1.0

_selc_np = np.zeros((LANES_CONV, 4 * CAND_STRIDE), np.float32)
for _j in range(POOL):
    for _co in range(C_OUT):
        for _wp in range(WP):
            _selc_np[_co * W + POOL * _wp + _j,
                     CAND_STRIDE * _j + _co * WP + _wp] = 1.0


def _cnn_kernel(xr_ref, wky_ref, selc_ref, b96_ref, wf_ref, bf_ref, out_ref):
    bblk = out_ref.shape[0]
    rows = H * bblk
    xt = xr_ref[...].reshape(rows, LANES_IN)
    wky = wky_ref[...]

    zpad = jnp.zeros((bblk, LANES_CONV), jnp.float32)
    conv = jnp.dot(xt, wky[1], preferred_element_type=jnp.float32)
    z0 = jnp.dot(xt, wky[0], preferred_element_type=jnp.float32)
    conv = conv + jnp.concatenate([zpad, z0[:rows - bblk]], axis=0)
    z2 = jnp.dot(xt, wky[2], preferred_element_type=jnp.float32)
    conv = conv + jnp.concatenate([z2[bblk:], zpad], axis=0)

    r12 = jnp.concatenate(
        [jnp.maximum(
            jnp.maximum(conv[(4 * hp) * bblk:(4 * hp + 1) * bblk],
                        conv[(4 * hp + 1) * bblk:(4 * hp + 2) * bblk]),
            jnp.maximum(conv[(4 * hp + 2) * bblk:(4 * hp + 3) * bblk],
                        conv[(4 * hp + 3) * bblk:(4 * hp + 4) * bblk]))
         for hp in range(HP)], axis=0)

    cand = jnp.dot(r12, selc_ref[...], preferred_element_type=jnp.float32)
    pooled = jnp.maximum(
        jnp.maximum(cand[:, 0:POOLED],
                    cand[:, CAND_STRIDE:CAND_STRIDE + POOLED]),
        jnp.maximum(cand[:, 2 * CAND_STRIDE:2 * CAND_STRIDE + POOLED],
                    cand[:, 3 * CAND_STRIDE:3 * CAND_STRIDE + POOLED]))
    act = jnp.maximum(pooled + b96_ref[...], 0.0)

    logits = jnp.broadcast_to(bf_ref[...], (bblk, N_CLASSES))
    wf = wf_ref[...]
    for hp in range(HP):
        logits = logits + jnp.dot(act[hp * bblk:(hp + 1) * bblk], wf[hp],
                                  preferred_element_type=jnp.float32)
    out_ref[...] = logits


@functools.partial(jax.jit, static_argnames=("block_b",))
def _forward(x, conv_w, conv_b, fc_w, fc_b, block_b=128):
    B = x.shape[0]
    bblk = block_b
    while B % bblk:
        bblk //= 2
    n_steps = B // bblk

    xr = (x.astype(jnp.float32).transpose(2, 0, 1, 3)
          .reshape(H, B, LANES_IN).astype(jnp.bfloat16))

    s = jnp.asarray(_shift_np)
    cw4 = conv_w.astype(jnp.float32).transpose(2, 1, 3, 0)
    wky = (s[None, None, :, :, None, :]
           * cw4[:, :, :, None, :, None]).sum(axis=2)
    wky = wky.reshape(KS, LANES_IN, LANES_CONV).astype(jnp.bfloat16)

    b96 = jnp.repeat(conv_b.astype(jnp.float32), WP).reshape(1, POOLED)
    wf = (fc_w.astype(jnp.float32).reshape(N_CLASSES, C_OUT, HP, WP)
          .transpose(2, 1, 3, 0).reshape(HP, POOLED, N_CLASSES))
    bf2 = fc_b.astype(jnp.float32).reshape(1, N_CLASSES)

    return pl.pallas_call(
        _cnn_kernel,
        out_shape=jax.ShapeDtypeStruct((B, N_CLASSES), jnp.float32),
        grid=(n_steps,),
        in_specs=[
            pl.BlockSpec((H, bblk, LANES_IN), lambda s: (0, s, 0)),
            pl.BlockSpec((KS, LANES_IN, LANES_CONV), lambda s: (0, 0, 0)),
            pl.BlockSpec((LANES_CONV, 4 * CAND_STRIDE), lambda s: (0, 0)),
            pl.BlockSpec((1, POOLED), lambda s: (0, 0)),
            pl.BlockSpec((HP, POOLED, N_CLASSES), lambda s: (0, 0, 0)),
            pl.BlockSpec((1, N_CLASSES), lambda s: (0, 0)),
        ],
        out_specs=pl.BlockSpec((bblk, N_CLASSES), lambda s: (s, 0)),
        compiler_params=pltpu.CompilerParams(
            dimension_semantics=("parallel",)),
    )(xr, wky, jnp.asarray(_selc_np), b96, wf, bf2)


def kernel(x, conv_w, conv_b, fc_w, fc_b):
    return _forward(x, conv_w, conv_b, fc_w, fc_b)
